# baseline (speedup 1.0000x reference)
import functools

import jax
import jax.numpy as jnp
from jax import lax
from jax.experimental import pallas as pl
from jax.experimental.pallas import tpu as pltpu
from jax.experimental.pallas import tpu_sc as plsc

N_LEVEL = 26
N_EMB = 100000
D_EMB = 32
BATCH = 16384

NUM_CORES = 2
NUM_SUBCORES = 16
NW = NUM_CORES * NUM_SUBCORES
BPW = BATCH // NW
IPG = 128
NG = BPW // IPG
LANES = 16


def _emb_kernel(idx_hbm, tab_hbm, out_hbm, idx_v, rows_v, sem):
    wid = lax.axis_index("s") * NUM_CORES + lax.axis_index("c")

    def level_body(l, carry):
        row0 = (l * NW + wid) * NG
        pltpu.sync_copy(idx_hbm.at[pl.ds(row0, NG)], idx_v)
        off = jnp.full((LANES,), l * N_EMB, jnp.int32)
        for j in range(NG):
            for i in range(IPG // LANES):
                sl = pl.ds(i * LANES, LANES)
                idx_v[j, sl] = idx_v[j, sl] + off
        copies = []
        for j in range(NG):
            copies.append(
                pltpu.async_copy(
                    tab_hbm.at[idx_v.at[j]],
                    rows_v.at[pl.ds(j * IPG, IPG)],
                    sem,
                )
            )
        for c in copies:
            c.wait()
        base = l * BATCH + wid * BPW
        pltpu.sync_copy(rows_v, out_hbm.at[pl.ds(base, BPW)])
        return carry

    lax.fori_loop(0, N_LEVEL, level_body, 0)


def kernel(idx, weight):
    idx_rows = idx.astype(jnp.int32).reshape(N_LEVEL * NW * NG, IPG)
    table = weight.reshape(N_LEVEL * N_EMB, D_EMB)

    mesh = plsc.VectorSubcoreMesh(core_axis_name="c", subcore_axis_name="s")
    run = functools.partial(
        pl.kernel,
        mesh=mesh,
        compiler_params=pltpu.CompilerParams(use_tc_tiling_on_sc=False),
        out_type=jax.ShapeDtypeStruct((N_LEVEL * BATCH, D_EMB), jnp.float32),
        scratch_types=[
            pltpu.VMEM((NG, IPG), jnp.int32),
            pltpu.VMEM((BPW, D_EMB), jnp.float32),
            pltpu.SemaphoreType.DMA,
        ],
    )(_emb_kernel)
    out = run(idx_rows, table)
    return out.reshape(N_LEVEL, BATCH, D_EMB)


# speedup vs baseline: 1.0207x; 1.0207x over previous
import functools

import jax
import jax.numpy as jnp
from jax import lax
from jax.experimental import pallas as pl
from jax.experimental.pallas import tpu as pltpu
from jax.experimental.pallas import tpu_sc as plsc

N_LEVEL = 26
N_EMB = 100000
D_EMB = 32
BATCH = 16384

NUM_CORES = 2
NUM_SUBCORES = 16
NW = NUM_CORES * NUM_SUBCORES
BPW = BATCH // NW
IPG = 128
NG = BPW // IPG
NBUF = 4


def _emb_kernel(idx_hbm, tab_hbm, out_hbm, idx_v, rows_v, sem_g, sem_o):
    wid = lax.axis_index("s") * NUM_CORES + lax.axis_index("c")

    def stage(l):
        b = l % NBUF
        row0 = (l * NW + wid) * NG
        pltpu.sync_copy(idx_hbm.at[pl.ds(row0, NG)], idx_v.at[b])

    def fire_gathers(l):
        b = l % NBUF
        return [
            pltpu.async_copy(
                tab_hbm.at[l].at[idx_v.at[b].at[j]],
                rows_v.at[b].at[pl.ds(j * IPG, IPG)],
                sem_g.at[b],
            )
            for j in range(NG)
        ]

    def fire_out(l):
        b = l % NBUF
        base = l * BATCH + wid * BPW
        return pltpu.async_copy(rows_v.at[b], out_hbm.at[pl.ds(base, BPW)],
                                sem_o.at[b])

    gathers = {}
    outs = {}
    for l in range(min(NBUF, N_LEVEL)):
        stage(l)
        gathers[l] = fire_gathers(l)
    for l in range(N_LEVEL):
        for c in gathers.pop(l):
            c.wait()
        outs[l] = fire_out(l)
        nl = l + NBUF
        if nl < N_LEVEL:
            stage(nl)
            outs[l].wait()
            gathers[nl] = fire_gathers(nl)
    for l in range(max(0, N_LEVEL - NBUF), N_LEVEL):
        if l in outs:
            outs[l].wait()


def kernel(idx, weight):
    idx_rows = idx.astype(jnp.int32).reshape(N_LEVEL * NW * NG, IPG)

    mesh = plsc.VectorSubcoreMesh(core_axis_name="c", subcore_axis_name="s")
    run = functools.partial(
        pl.kernel,
        mesh=mesh,
        compiler_params=pltpu.CompilerParams(use_tc_tiling_on_sc=False),
        out_type=jax.ShapeDtypeStruct((N_LEVEL * BATCH, D_EMB), jnp.float32),
        scratch_types=[
            pltpu.VMEM((NBUF, NG, IPG), jnp.int32),
            pltpu.VMEM((NBUF, BPW, D_EMB), jnp.float32),
            pltpu.SemaphoreType.DMA((NBUF,)),
            pltpu.SemaphoreType.DMA((NBUF,)),
        ],
    )(_emb_kernel)
    out = run(idx_rows, weight)
    return out.reshape(N_LEVEL, BATCH, D_EMB)


# speedup vs baseline: 5.1545x; 5.0501x over previous
import functools

import jax
import jax.numpy as jnp
from jax import lax
from jax.experimental import pallas as pl
from jax.experimental.pallas import tpu as pltpu
from jax.experimental.pallas import tpu_sc as plsc

N_LEVEL = 26
N_EMB = 100000
D_EMB = 32
BATCH = 16384

NUM_CORES = 2
NUM_SUBCORES = 16
NW = NUM_CORES * NUM_SUBCORES
ROWS = N_LEVEL * D_EMB
RPW = ROWS // NW
LANES = 16
HALF = BATCH // 2
GROUPS = HALF // (LANES * 8)


def _emb_kernel(idx_hbm, tab_hbm, out_hbm, idx_v, row_v, out_v):
    wid = lax.axis_index("s") * NUM_CORES + lax.axis_index("c")

    def row_body(j, l_prev):
        r = wid * RPW + j
        l = lax.shift_right_logical(r, 5)
        d = lax.bitwise_and(r, 31)

        @pl.when(l != l_prev)
        def _():
            pltpu.sync_copy(idx_hbm.at[l], idx_v)

        pltpu.sync_copy(tab_hbm.at[l, d], row_v)

        for h in range(2):
            def gather_body(g, c):
                base = g * (LANES * 8)
                for k in range(8):
                    sl = pl.ds(h * HALF + base + k * LANES, LANES)
                    iv = idx_v[sl]
                    out_v[pl.ds(base + k * LANES, LANES)] = (
                        plsc.load_gather(row_v, [iv]))
                return c
            lax.fori_loop(0, GROUPS, gather_body, 0)
            pltpu.sync_copy(out_v, out_hbm.at[l, d, pl.ds(h * HALF, HALF)])
        return l

    lax.fori_loop(0, RPW, row_body, jnp.int32(-1))


def kernel(idx, weight):
    tab_t = jnp.transpose(weight, (0, 2, 1))

    mesh = plsc.VectorSubcoreMesh(core_axis_name="c", subcore_axis_name="s")
    run = functools.partial(
        pl.kernel,
        mesh=mesh,
        compiler_params=pltpu.CompilerParams(needs_layout_passes=False),
        out_type=jax.ShapeDtypeStruct((N_LEVEL, D_EMB, BATCH), jnp.float32),
        scratch_types=[
            pltpu.VMEM((BATCH,), jnp.int32),
            pltpu.VMEM((N_EMB,), jnp.float32),
            pltpu.VMEM((HALF,), jnp.float32),
        ],
    )(_emb_kernel)
    out_t = run(idx.astype(jnp.int32), tab_t)
    return jnp.transpose(out_t, (0, 2, 1))
